# baseline (speedup 1.0000x reference)
import jax
import jax.numpy as jnp
from jax import lax
from jax.experimental import pallas as pl
from jax.experimental.pallas import tpu as pltpu
from jax.experimental.pallas import tpu_sc as plsc

F, N, P, W, B = 3, 64, 131072, 50, 1024
FN = F * N
WIN = 64
WS = 65
GBS = 17
BPS = 33
NW = 32
BPT = B // NW
GB = 16
WN = W * N
NS = 32
SLAB = P // NS
SLAB_SHIFT = 12


def _sc_body(cf_hbm, idx_hbm, xt_hbm, yt_hbm,
             idxall, wbufF, xtbuf, ybuf, rbufG,
             sem_w0, sem_w1, sem_x0, sem_x1):
    wid = lax.axis_index("s") * 2 + lax.axis_index("c")
    base_b = wid * BPT

    pltpu.sync_copy(idx_hbm, idxall.at[pl.ds(0, B)])

    lanev = lax.iota(jnp.int32, 16)
    zeros = jnp.zeros((16,), jnp.int32)
    wsems = (sem_w0, sem_w1)
    xsems = (sem_x0, sem_x1)

    def win_copy(f, b, buf):
        i = idxall[pl.ds(b, 16)][0]
        a = jnp.minimum((i // 8) * 8, P - WIN)
        return pltpu.make_async_copy(
            cf_hbm.at[pl.ds(f * N, N), pl.ds(a, WIN)],
            wbufF.at[buf, :, pl.ds(0, WIN)], wsems[buf])

    bt0 = wid // 4
    bi0 = (wid % 4) * BPT

    def xt_copy(seg, buf):
        g, f = divmod(seg, 3)
        return pltpu.make_async_copy(
            xtbuf.at[0, :, :, :, pl.ds(0, GB)],
            xt_hbm.at[f, :, :, bt0, :, pl.ds(bi0 + g * GB, GB)],
            xsems[0])

    win_copy(0, base_b, 0).start()
    for seg in range(6):
        g, f = divmod(seg, 3)
        grp = base_b + g * GB
        xb = 0
        if seg >= 1:
            xt_copy(seg - 1, 0).wait()

        def pair2(t, carry, f=f, grp=grp, g=g, xb=xb):
            for k in (0, 1):
                jj = 2 * t + k
                b = grp + jj

                if k == 0:
                    win_copy(f, b + 1, 1 - k).start()
                else:
                    @pl.when(t < 7)
                    def _(f=f, b=b, k=k):
                        win_copy(f, b + 1, 1 - k).start()

                win_copy(f, b, k).wait()
                i = idxall[pl.ds(b, 16)][0]
                a = jnp.minimum((i // 8) * 8, P - WIN)
                o = i - a
                kv = zeros + k
                xbv = zeros + xb
                colv = zeros + jj
                nrows = tuple(lanev + 16 * c for c in range(4))
                ntv = tuple((lanev + 16 * c) // 8 for c in range(4))
                niv = lanev % 8

                if f == 0:
                    for c in range(4):
                        d = plsc.load_gather(
                            wbufF, [kv, nrows[c], zeros + (o + W - 1)])
                        rbufG[jj, pl.ds(16 * c, 16)] = 1.0 / d

                rn = tuple(rbufG[jj, pl.ds(16 * c, 16)] for c in range(4))

                def per_w(t2, c2, o=o, jj=jj, kv=kv, xbv=xbv, colv=colv,
                          nrows=nrows, rn=rn, ntv=ntv, niv=niv):
                    w = t2 * 2
                    for dw in range(2):
                        colw = zeros + (o + w + dw)
                        wv = zeros + (w + dw)
                        for c in range(4):
                            gv = plsc.load_gather(
                                wbufF, [kv, nrows[c], colw])
                            plsc.store_scatter(
                                xtbuf, [xbv, wv, ntv[c], niv, colv],
                                gv * rn[c])
                    return c2

                lax.fori_loop(0, W // 2, per_w, 0)

                fv = zeros + f
                ycol = zeros + (g * GB + jj)
                for c in range(4):
                    gv = plsc.load_gather(
                        wbufF, [kv, nrows[c], zeros + (o + W)])
                    plsc.store_scatter(
                        ybuf, [fv, nrows[c], ycol], gv * rn[c])
            return carry

        lax.fori_loop(0, 8, pair2, 0)
        if seg < 5:
            ng, nf = divmod(seg + 1, 3)
            win_copy(nf, base_b + ng * GB, 0).start()
        xt_copy(seg, xb).start()

    xt_copy(5, 0).wait()
    pltpu.sync_copy(ybuf.at[:, :, pl.ds(0, BPT)],
                    yt_hbm.at[:, :, pl.ds(base_b, BPT)])


def _sc_xy(cf2, index):
    mesh = plsc.VectorSubcoreMesh(core_axis_name="c", subcore_axis_name="s")
    fn = pl.kernel(
        _sc_body,
        out_type=[
            jax.ShapeDtypeStruct((F, W, 8, 8, 8, 128), jnp.float32),
            jax.ShapeDtypeStruct((F, N, B), jnp.float32),
        ],
        mesh=mesh,
        compiler_params=pltpu.CompilerParams(
            use_tc_tiling_on_sc=False, needs_layout_passes=False),
        scratch_types=[
            pltpu.VMEM((B + 16,), jnp.int32),
            pltpu.VMEM((2, N, WS), jnp.float32),
            pltpu.VMEM((1, W, 8, 8, GBS), jnp.float32),
            pltpu.VMEM((F, N, BPS), jnp.float32),
            pltpu.VMEM((GB, N + 16), jnp.float32),
            pltpu.SemaphoreType.DMA,
            pltpu.SemaphoreType.DMA,
            pltpu.SemaphoreType.DMA,
            pltpu.SemaphoreType.DMA,
        ],
    )
    return fn(cf2, index)


def _tc_body(idx_ref, wt_ref, pvmt_ref, outt_ref, lwt_ref,
             perm_s, st_s, perm_g, st_g, cnt, cur):
    s = pl.program_id(0)

    @pl.when(s == 0)
    def _build_routing():
        for perm, st, shift_src in ((perm_s, st_s, 0), (perm_g, st_g, 1)):
            def zero(k, c):
                cnt[k] = 0
                return c
            lax.fori_loop(0, NS, zero, 0)

            def count(b, c):
                sl = (idx_ref[b] - shift_src) >> SLAB_SHIFT
                cnt[sl] = cnt[sl] + 1
                return c
            lax.fori_loop(0, B, count, 0)

            st[0] = 0

            def prefix(k, c):
                st[k + 1] = st[k] + cnt[k]
                cur[k] = st[k]
                return c
            lax.fori_loop(0, NS, prefix, 0)

            def place(b, c):
                sl = (idx_ref[b] - shift_src) >> SLAB_SHIFT
                perm[cur[sl]] = b
                cur[sl] = cur[sl] + 1
                return c
            lax.fori_loop(0, B, place, 0)

    outt_ref[...] = pvmt_ref[...]
    base = s * SLAB

    def scat(k, c):
        b = perm_s[k]
        outt_ref[pl.ds(idx_ref[b] - base, 1), :] = wt_ref[pl.ds(b, 1), :]
        return c

    lax.fori_loop(st_s[s], st_s[s + 1], scat, 0)

    def gath(k, c):
        b = perm_g[k]
        lwt_ref[pl.ds(b, 1), :] = pvmt_ref[pl.ds(idx_ref[b] - 1 - base, 1), :]
        return c

    lax.fori_loop(st_g[s], st_g[s + 1], gath, 0)


def _tc_pvm(pvm, index, w):
    return pl.pallas_call(
        _tc_body,
        grid=(NS,),
        in_specs=[
            pl.BlockSpec(memory_space=pltpu.SMEM),
            pl.BlockSpec((B, N), lambda i: (0, 0)),
            pl.BlockSpec((SLAB, N), lambda i: (i, 0)),
        ],
        out_specs=[
            pl.BlockSpec((SLAB, N), lambda i: (i, 0)),
            pl.BlockSpec((B, N), lambda i: (0, 0)),
        ],
        out_shape=[
            jax.ShapeDtypeStruct((P, N), jnp.float32),
            jax.ShapeDtypeStruct((B, N), jnp.float32),
        ],
        scratch_shapes=[
            pltpu.SMEM((B,), jnp.int32),
            pltpu.SMEM((NS + 1,), jnp.int32),
            pltpu.SMEM((B,), jnp.int32),
            pltpu.SMEM((NS + 1,), jnp.int32),
            pltpu.SMEM((NS,), jnp.int32),
            pltpu.SMEM((NS,), jnp.int32),
        ],
        compiler_params=pltpu.CompilerParams(
            dimension_semantics=("arbitrary",)),
    )(index, w, pvm)


def kernel(coin_features, pvm, index, w):
    cf2 = coin_features.reshape(FN, P)
    xt6, yt3 = _sc_xy(cf2, index)
    new_pvm, last_w = _tc_pvm(pvm, index, w)
    X = xt6.transpose(3, 5, 0, 2, 4, 1).reshape(B, F, N, W)
    y = yt3.transpose(2, 0, 1)
    return X, y, last_w, new_pvm


# speedup vs baseline: 1.0008x; 1.0008x over previous
import jax
import jax.numpy as jnp
from jax import lax
from jax.experimental import pallas as pl
from jax.experimental.pallas import tpu as pltpu
from jax.experimental.pallas import tpu_sc as plsc

F, N, P, W, B = 3, 64, 131072, 50, 1024
FN = F * N
WIN = 64
WS = 65
GBS = 17
BPS = 33
NW = 32
BPT = B // NW
GB = 16
WN = W * N
NS = 32
SLAB = P // NS
SLAB_SHIFT = 12


def _sc_body(cf_hbm, idx_hbm, xt_hbm, yt_hbm,
             idxall, wbufF, xtbuf, ybuf, rbufG,
             sem_w0, sem_w1, sem_x0, sem_x1):
    wid = lax.axis_index("s") * 2 + lax.axis_index("c")
    base_b = wid * BPT

    pltpu.sync_copy(idx_hbm, idxall.at[pl.ds(0, B)])

    lanev = lax.iota(jnp.int32, 16)
    zeros = jnp.zeros((16,), jnp.int32)
    wsems = (sem_w0, sem_w1)
    xsems = (sem_x0, sem_x1)

    def win_copy(f, b, buf):
        i = idxall[pl.ds(b, 16)][0]
        a = jnp.minimum((i // 8) * 8, P - WIN)
        return pltpu.make_async_copy(
            cf_hbm.at[pl.ds(f * N, N), pl.ds(a, WIN)],
            wbufF.at[buf, :, pl.ds(0, WIN)], wsems[buf])

    bt0 = wid // 4
    bi0 = (wid % 4) * BPT

    def xt_copy(seg, buf):
        g, f = divmod(seg, 3)
        return pltpu.make_async_copy(
            xtbuf.at[0, :, :, :, pl.ds(0, GB)],
            xt_hbm.at[f, :, :, bt0, :, pl.ds(bi0 + g * GB, GB)],
            xsems[0])

    win_copy(0, base_b, 0).start()
    for seg in range(6):
        g, f = divmod(seg, 3)
        grp = base_b + g * GB
        xb = 0
        if seg >= 1:
            xt_copy(seg - 1, 0).wait()

        def pair2(t, carry, f=f, grp=grp, g=g, xb=xb):
            for k in (0, 1):
                jj = 2 * t + k
                b = grp + jj

                if k == 0:
                    win_copy(f, b + 1, 1 - k).start()
                else:
                    @pl.when(t < 7)
                    def _(f=f, b=b, k=k):
                        win_copy(f, b + 1, 1 - k).start()

                win_copy(f, b, k).wait()
                i = idxall[pl.ds(b, 16)][0]
                a = jnp.minimum((i // 8) * 8, P - WIN)
                o = i - a
                kv = zeros + k
                xbv = zeros + xb
                colv = zeros + jj
                nrows = tuple(lanev + 16 * c for c in range(4))
                ntv = tuple((lanev + 16 * c) // 8 for c in range(4))
                niv = lanev % 8

                if f == 0:
                    for c in range(4):
                        d = plsc.load_gather(
                            wbufF, [kv, nrows[c], zeros + (o + W - 1)])
                        rbufG[jj, pl.ds(16 * c, 16)] = 1.0 / d

                rn = tuple(rbufG[jj, pl.ds(16 * c, 16)] for c in range(4))

                def per_w(w, c2, o=o, jj=jj, kv=kv, xbv=xbv, colv=colv,
                          nrows=nrows, rn=rn, ntv=ntv, niv=niv):
                    colw = zeros + (o + w)
                    wv = zeros + w
                    for c in range(4):
                        gv = plsc.load_gather(wbufF, [kv, nrows[c], colw])
                        plsc.store_scatter(
                            xtbuf, [xbv, wv, ntv[c], niv, colv], gv * rn[c])
                    return c2

                lax.fori_loop(0, W, per_w, 0)

                fv = zeros + f
                ycol = zeros + (g * GB + jj)
                for c in range(4):
                    gv = plsc.load_gather(
                        wbufF, [kv, nrows[c], zeros + (o + W)])
                    plsc.store_scatter(
                        ybuf, [fv, nrows[c], ycol], gv * rn[c])
            return carry

        lax.fori_loop(0, 8, pair2, 0)
        if seg < 5:
            ng, nf = divmod(seg + 1, 3)
            win_copy(nf, base_b + ng * GB, 0).start()
        xt_copy(seg, xb).start()

    xt_copy(5, 0).wait()
    pltpu.sync_copy(ybuf.at[:, :, pl.ds(0, BPT)],
                    yt_hbm.at[:, :, pl.ds(base_b, BPT)])


def _sc_xy(cf2, index):
    mesh = plsc.VectorSubcoreMesh(core_axis_name="c", subcore_axis_name="s")
    fn = pl.kernel(
        _sc_body,
        out_type=[
            jax.ShapeDtypeStruct((F, W, 8, 8, 8, 128), jnp.float32),
            jax.ShapeDtypeStruct((F, N, B), jnp.float32),
        ],
        mesh=mesh,
        compiler_params=pltpu.CompilerParams(
            use_tc_tiling_on_sc=False, needs_layout_passes=False),
        scratch_types=[
            pltpu.VMEM((B + 16,), jnp.int32),
            pltpu.VMEM((2, N, WS), jnp.float32),
            pltpu.VMEM((1, W, 8, 8, GBS), jnp.float32),
            pltpu.VMEM((F, N, BPS), jnp.float32),
            pltpu.VMEM((GB, N + 16), jnp.float32),
            pltpu.SemaphoreType.DMA,
            pltpu.SemaphoreType.DMA,
            pltpu.SemaphoreType.DMA,
            pltpu.SemaphoreType.DMA,
        ],
    )
    return fn(cf2, index)


def _tc_body(idx_ref, wt_ref, pvmt_ref, outt_ref, lwt_ref,
             perm_s, st_s, perm_g, st_g, cnt, cur):
    s = pl.program_id(0)

    @pl.when(s == 0)
    def _build_routing():
        for perm, st, shift_src in ((perm_s, st_s, 0), (perm_g, st_g, 1)):
            def zero(k, c):
                cnt[k] = 0
                return c
            lax.fori_loop(0, NS, zero, 0)

            def count(b, c):
                sl = (idx_ref[b] - shift_src) >> SLAB_SHIFT
                cnt[sl] = cnt[sl] + 1
                return c
            lax.fori_loop(0, B, count, 0)

            st[0] = 0

            def prefix(k, c):
                st[k + 1] = st[k] + cnt[k]
                cur[k] = st[k]
                return c
            lax.fori_loop(0, NS, prefix, 0)

            def place(b, c):
                sl = (idx_ref[b] - shift_src) >> SLAB_SHIFT
                perm[cur[sl]] = b
                cur[sl] = cur[sl] + 1
                return c
            lax.fori_loop(0, B, place, 0)

    outt_ref[...] = pvmt_ref[...]
    base = s * SLAB

    def scat(k, c):
        b = perm_s[k]
        outt_ref[pl.ds(idx_ref[b] - base, 1), :] = wt_ref[pl.ds(b, 1), :]
        return c

    lax.fori_loop(st_s[s], st_s[s + 1], scat, 0)

    def gath(k, c):
        b = perm_g[k]
        lwt_ref[pl.ds(b, 1), :] = pvmt_ref[pl.ds(idx_ref[b] - 1 - base, 1), :]
        return c

    lax.fori_loop(st_g[s], st_g[s + 1], gath, 0)


def _tc_pvm(pvm, index, w):
    return pl.pallas_call(
        _tc_body,
        grid=(NS,),
        in_specs=[
            pl.BlockSpec(memory_space=pltpu.SMEM),
            pl.BlockSpec((B, N), lambda i: (0, 0)),
            pl.BlockSpec((SLAB, N), lambda i: (i, 0)),
        ],
        out_specs=[
            pl.BlockSpec((SLAB, N), lambda i: (i, 0)),
            pl.BlockSpec((B, N), lambda i: (0, 0)),
        ],
        out_shape=[
            jax.ShapeDtypeStruct((P, N), jnp.float32),
            jax.ShapeDtypeStruct((B, N), jnp.float32),
        ],
        scratch_shapes=[
            pltpu.SMEM((B,), jnp.int32),
            pltpu.SMEM((NS + 1,), jnp.int32),
            pltpu.SMEM((B,), jnp.int32),
            pltpu.SMEM((NS + 1,), jnp.int32),
            pltpu.SMEM((NS,), jnp.int32),
            pltpu.SMEM((NS,), jnp.int32),
        ],
        compiler_params=pltpu.CompilerParams(
            dimension_semantics=("arbitrary",)),
    )(index, w, pvm)


def kernel(coin_features, pvm, index, w):
    cf2 = coin_features.reshape(FN, P)
    xt6, yt3 = _sc_xy(cf2, index)
    new_pvm, last_w = _tc_pvm(pvm, index, w)
    X = xt6.transpose(3, 5, 0, 2, 4, 1).reshape(B, F, N, W)
    y = yt3.transpose(2, 0, 1)
    return X, y, last_w, new_pvm
